# baseline (speedup 1.0000x reference)
import functools

import jax
import jax.numpy as jnp
from jax import lax
from jax.experimental import pallas as pl
from jax.experimental.pallas import tpu as pltpu
from jax.experimental.pallas import tpu_sc as plsc

N = 10000
D = 128
E = 320000
NC = 2
NS = 16
CH = 128
NCH = 80
E_PAD = NC * NS * NCH * CH
N_PAD = 10240
RPT = N_PAD // NS
DCH = 2 * NCH


def _sc_mesh():
    return plsc.VectorSubcoreMesh(core_axis_name="c", subcore_axis_name="s")


def _sc_degree(idx2, zeros128, ones128):
    @functools.partial(
        pl.kernel,
        mesh=_sc_mesh(),
        out_type=jax.ShapeDtypeStruct((2 * N_PAD, D), jnp.float32),
        scratch_types=[
            pltpu.VMEM((DCH, CH), jnp.int32),
            pltpu.VMEM((CH, D), jnp.float32),
            pltpu.VMEM_SHARED((N_PAD, D), jnp.float32),
        ],
    )
    def k(idx_hbm, z_hbm, o_hbm, deg_hbm, idx_all, ones_v, acc):
        c = lax.axis_index("c")
        s = lax.axis_index("s")
        pltpu.sync_copy(z_hbm, acc.at[pl.ds(s * RPT, RPT)])
        pltpu.sync_copy(o_hbm, ones_v)
        pltpu.sync_copy(idx_hbm.at[c * NS + s], idx_all)
        plsc.subcore_barrier()

        def body(g, carry):
            pltpu.sync_copy(ones_v, acc.at[idx_all.at[g]], add=True)
            return carry

        lax.fori_loop(0, DCH, body, 0)
        plsc.subcore_barrier()
        pltpu.sync_copy(
            acc.at[pl.ds(s * RPT, RPT)],
            deg_hbm.at[pl.ds(c * N_PAD + s * RPT, RPT)],
        )

    return k(idx2, zeros128, ones128)


def _sc_scatter(hn, pair_r, zeros128):
    @functools.partial(
        pl.kernel,
        mesh=_sc_mesh(),
        out_type=jax.ShapeDtypeStruct((2 * N_PAD, D), jnp.float32),
        scratch_types=[
            pltpu.VMEM((2, CH), jnp.int32),
            pltpu.VMEM((2, CH), jnp.int32),
            pltpu.VMEM((CH, D), jnp.float32),
            pltpu.VMEM((CH, D), jnp.float32),
            pltpu.SemaphoreType.DMA,
            pltpu.SemaphoreType.DMA,
            pltpu.SemaphoreType.DMA,
            pltpu.SemaphoreType.DMA,
            pltpu.VMEM_SHARED((N_PAD, D), jnp.float32),
        ],
    )
    def k(hn_hbm, pair_hbm, z_hbm, out_hbm,
          ibuf0, ibuf1, rbuf0, rbuf1, isem0, isem1, rsem0, rsem1, acc):
        c = lax.axis_index("c")
        s = lax.axis_index("s")
        wid = s * NC + c
        pltpu.sync_copy(z_hbm, acc.at[pl.ds(s * RPT, RPT)])
        plsc.subcore_barrier()

        def idx_start(g, ibuf, isem):
            pltpu.async_copy(pair_hbm.at[wid, g], ibuf, isem)

        def idx_wait(ibuf, isem):
            pltpu.make_async_copy(pair_hbm.at[wid, 0], ibuf, isem).wait()

        def gather_start(ibuf, rbuf, rsem):
            pltpu.async_copy(hn_hbm.at[ibuf.at[0]], rbuf, rsem)

        def gather_wait(ibuf, rbuf, rsem):
            pltpu.make_async_copy(hn_hbm.at[ibuf.at[0]], rbuf, rsem).wait()

        def scat(ibuf, rbuf):
            pltpu.sync_copy(rbuf, acc.at[ibuf.at[1]], add=True)

        idx_start(0, ibuf0, isem0)
        idx_start(1, ibuf1, isem1)
        idx_wait(ibuf0, isem0)
        gather_start(ibuf0, rbuf0, rsem0)

        def body(i, carry):
            g = 2 * i
            idx_wait(ibuf1, isem1)
            gather_start(ibuf1, rbuf1, rsem1)
            gather_wait(ibuf0, rbuf0, rsem0)
            scat(ibuf0, rbuf0)
            idx_start(g + 2, ibuf0, isem0)
            idx_wait(ibuf0, isem0)
            gather_start(ibuf0, rbuf0, rsem0)
            gather_wait(ibuf1, rbuf1, rsem1)
            scat(ibuf1, rbuf1)
            idx_start(g + 3, ibuf1, isem1)
            return carry

        lax.fori_loop(0, NCH // 2 - 1, body, 0)
        idx_wait(ibuf1, isem1)
        gather_start(ibuf1, rbuf1, rsem1)
        gather_wait(ibuf0, rbuf0, rsem0)
        scat(ibuf0, rbuf0)
        gather_wait(ibuf1, rbuf1, rsem1)
        scat(ibuf1, rbuf1)

        plsc.subcore_barrier()
        pltpu.sync_copy(
            acc.at[pl.ds(s * RPT, RPT)],
            out_hbm.at[pl.ds(c * N_PAD + s * RPT, RPT)],
        )

    return k(hn, pair_r, zeros128)


def _first_body(x_ref, w_ref, deg_ref, o_ref):
    nsrc = lax.rsqrt(jnp.maximum(deg_ref[:, 0:1], 1.0))
    o_ref[...] = (
        jnp.dot(x_ref[...], w_ref[...], preferred_element_type=jnp.float32)
        * nsrc
    )


def _tc_first(x, w, deg_out):
    return pl.pallas_call(
        _first_body,
        out_shape=jax.ShapeDtypeStruct((N, D), jnp.float32),
    )(x, w, deg_out)


def _mid_body(p0_ref, p1_ref, din_ref, dout_ref, b_ref, w_ref, o_ref):
    ndst = lax.rsqrt(jnp.maximum(din_ref[:, 0:1], 1.0))
    t = jnp.maximum((p0_ref[...] + p1_ref[...]) * ndst + b_ref[...], 0.0)
    nsrc = lax.rsqrt(jnp.maximum(dout_ref[:, 0:1], 1.0))
    o_ref[...] = (
        jnp.dot(t, w_ref[...], preferred_element_type=jnp.float32) * nsrc
    )


def _tc_mid(p0, p1, deg_in, deg_out, b, w):
    return pl.pallas_call(
        _mid_body,
        out_shape=jax.ShapeDtypeStruct((N, D), jnp.float32),
    )(p0, p1, deg_in, deg_out, b, w)


def _final_body(p0_ref, p1_ref, din_ref, b_ref, wm_ref, wa_ref, o_ref):
    ndst = lax.rsqrt(jnp.maximum(din_ref[:, 0:1], 1.0))
    t = jnp.maximum((p0_ref[...] + p1_ref[...]) * ndst + b_ref[...], 0.0)
    mx = jnp.max(t, axis=0, keepdims=True)
    sm = jnp.sum(t, axis=0, keepdims=True)
    o_ref[...] = wm_ref[...] * mx + (wa_ref[...] / N) * sm


def _tc_final(p0, p1, deg_in, b, wm, wa):
    return pl.pallas_call(
        _final_body,
        out_shape=jax.ShapeDtypeStruct((1, D), jnp.float32),
    )(p0, p1, deg_in, b, wm, wa)


def kernel(features, edge_index, W0, b0, W1, b1, W2, b2, pool_weight):
    src = edge_index[0]
    dst = edge_index[1]
    npad = E_PAD - E
    pad_sink = jnp.full((npad,), N, dtype=jnp.int32)
    pad_zero = jnp.zeros((npad,), dtype=jnp.int32)
    src_r = jnp.concatenate([src, pad_zero]).reshape(NC * NS, NCH, 1, CH)
    dst_r = jnp.concatenate([dst, pad_sink]).reshape(NC * NS, NCH, 1, CH)
    pair_r = jnp.concatenate([src_r, dst_r], axis=2)
    idx2 = jnp.concatenate(
        [src, pad_sink, dst, pad_sink]).reshape(2 * NS, DCH, CH)
    ones128 = jnp.ones((CH, D), jnp.float32)
    zeros128 = jnp.zeros((RPT, D), jnp.float32)

    deg = _sc_degree(idx2, zeros128, ones128)
    deg_out = deg[:N]
    deg_in = deg[N_PAD:N_PAD + N]

    hn = _tc_first(features, W0, deg_out)
    part = _sc_scatter(hn, pair_r, zeros128)
    p0, p1 = part[:N], part[N_PAD:N_PAD + N]
    hn = _tc_mid(p0, p1, deg_in, deg_out, b0.reshape(1, D), W1)
    part = _sc_scatter(hn, pair_r, zeros128)
    p0, p1 = part[:N], part[N_PAD:N_PAD + N]
    hn = _tc_mid(p0, p1, deg_in, deg_out, b1.reshape(1, D), W2)
    part = _sc_scatter(hn, pair_r, zeros128)
    p0, p1 = part[:N], part[N_PAD:N_PAD + N]

    w = jax.nn.softmax(pool_weight, axis=0)
    return _tc_final(
        p0, p1, deg_in, b2.reshape(1, D),
        w[0].reshape(1, 1), w[1].reshape(1, 1),
    )


# speedup vs baseline: 1.0003x; 1.0003x over previous
import functools

import jax
import jax.numpy as jnp
from jax import lax
from jax.experimental import pallas as pl
from jax.experimental.pallas import tpu as pltpu
from jax.experimental.pallas import tpu_sc as plsc

N = 10000
D = 128
E = 320000
NC = 2
NS = 16
CH = 128
NCH = 80
E_PAD = NC * NS * NCH * CH
N_PAD = 10112
RPT = N_PAD // NS
DCH = 160
CHD = 128


def _sc_mesh():
    return plsc.VectorSubcoreMesh(core_axis_name="c", subcore_axis_name="s")


def _sc_degree(idx2, zeros128, ones128):
    @functools.partial(
        pl.kernel,
        mesh=_sc_mesh(),
        out_type=jax.ShapeDtypeStruct((2 * N_PAD, D), jnp.float32),
        scratch_types=[
            pltpu.VMEM((DCH, CHD), jnp.int32),
            pltpu.VMEM((CHD, D), jnp.float32),
            pltpu.VMEM_SHARED((N_PAD, D), jnp.float32),
        ],
    )
    def k(idx_hbm, z_hbm, o_hbm, deg_hbm, idx_all, ones_v, acc):
        c = lax.axis_index("c")
        s = lax.axis_index("s")
        pltpu.sync_copy(z_hbm, acc.at[pl.ds(s * RPT, RPT)])
        pltpu.sync_copy(o_hbm, ones_v)
        pltpu.sync_copy(idx_hbm.at[c * NS + s], idx_all)
        plsc.subcore_barrier()

        def body(g, carry):
            pltpu.sync_copy(ones_v, acc.at[idx_all.at[g]], add=True)
            return carry

        lax.fori_loop(0, DCH, body, 0)
        plsc.subcore_barrier()
        pltpu.sync_copy(
            acc.at[pl.ds(s * RPT, RPT)],
            deg_hbm.at[pl.ds(c * N_PAD + s * RPT, RPT)],
        )

    return k(idx2, zeros128, ones128)


def _sc_scatter(hn, src_r, dst_r, zeros128):
    @functools.partial(
        pl.kernel,
        mesh=_sc_mesh(),
        out_type=jax.ShapeDtypeStruct((2 * N_PAD, D), jnp.float32),
        scratch_types=[
            pltpu.VMEM((NCH, CH), jnp.int32),
            pltpu.VMEM((CH,), jnp.int32),
            pltpu.VMEM((CH,), jnp.int32),
            pltpu.VMEM((CH, D), jnp.float32),
            pltpu.VMEM((CH, D), jnp.float32),
            pltpu.SemaphoreType.DMA,
            pltpu.SemaphoreType.DMA,
            pltpu.SemaphoreType.DMA,
            pltpu.SemaphoreType.DMA,
            pltpu.VMEM_SHARED((N_PAD, D), jnp.float32),
        ],
    )
    def k(hn_hbm, src_hbm, dst_hbm, z_hbm, out_hbm,
          didx, ibuf0, ibuf1, rbuf0, rbuf1,
          isem0, isem1, rsem0, rsem1, acc):
        c = lax.axis_index("c")
        s = lax.axis_index("s")
        wid = s * NC + c
        pltpu.sync_copy(z_hbm, acc.at[pl.ds(s * RPT, RPT)])
        pltpu.sync_copy(dst_hbm.at[wid], didx)
        plsc.subcore_barrier()

        def idx_start(g, ibuf, isem):
            pltpu.async_copy(src_hbm.at[wid, g], ibuf, isem)

        def idx_wait(ibuf, isem):
            pltpu.make_async_copy(src_hbm.at[wid, 0], ibuf, isem).wait()

        def gather_start(ibuf, rbuf, rsem):
            pltpu.async_copy(hn_hbm.at[ibuf], rbuf, rsem)

        def gather_wait(ibuf, rbuf, rsem):
            pltpu.make_async_copy(hn_hbm.at[ibuf], rbuf, rsem).wait()

        def scat(g, rbuf):
            pltpu.sync_copy(rbuf, acc.at[didx.at[g]], add=True)

        idx_start(0, ibuf0, isem0)
        idx_start(1, ibuf1, isem1)
        idx_wait(ibuf0, isem0)
        gather_start(ibuf0, rbuf0, rsem0)

        def body(i, carry):
            g = 2 * i
            idx_wait(ibuf1, isem1)
            gather_wait(ibuf0, rbuf0, rsem0)
            gather_start(ibuf1, rbuf1, rsem1)
            idx_start(g + 2, ibuf0, isem0)
            scat(g, rbuf0)
            idx_wait(ibuf0, isem0)
            gather_wait(ibuf1, rbuf1, rsem1)
            gather_start(ibuf0, rbuf0, rsem0)
            idx_start(g + 3, ibuf1, isem1)
            scat(g + 1, rbuf1)
            return carry

        lax.fori_loop(0, NCH // 2 - 1, body, 0)
        gl = NCH - 2
        idx_wait(ibuf1, isem1)
        gather_wait(ibuf0, rbuf0, rsem0)
        gather_start(ibuf1, rbuf1, rsem1)
        scat(gl, rbuf0)
        gather_wait(ibuf1, rbuf1, rsem1)
        scat(gl + 1, rbuf1)

        plsc.subcore_barrier()
        pltpu.sync_copy(
            acc.at[pl.ds(s * RPT, RPT)],
            out_hbm.at[pl.ds(c * N_PAD + s * RPT, RPT)],
        )

    return k(hn, src_r, dst_r, zeros128)


def _first_body(x_ref, w_ref, deg_ref, o_ref):
    nsrc = lax.rsqrt(jnp.maximum(deg_ref[:, 0:1], 1.0))
    o_ref[...] = (
        jnp.dot(x_ref[...], w_ref[...], preferred_element_type=jnp.float32)
        * nsrc
    )


def _tc_first(x, w, deg_out):
    return pl.pallas_call(
        _first_body,
        out_shape=jax.ShapeDtypeStruct((N, D), jnp.float32),
    )(x, w, deg_out)


def _mid_body(p0_ref, p1_ref, din_ref, dout_ref, b_ref, w_ref, o_ref):
    ndst = lax.rsqrt(jnp.maximum(din_ref[:, 0:1], 1.0))
    t = jnp.maximum((p0_ref[...] + p1_ref[...]) * ndst + b_ref[...], 0.0)
    nsrc = lax.rsqrt(jnp.maximum(dout_ref[:, 0:1], 1.0))
    o_ref[...] = (
        jnp.dot(t, w_ref[...], preferred_element_type=jnp.float32) * nsrc
    )


def _tc_mid(p0, p1, deg_in, deg_out, b, w):
    return pl.pallas_call(
        _mid_body,
        out_shape=jax.ShapeDtypeStruct((N, D), jnp.float32),
    )(p0, p1, deg_in, deg_out, b, w)


def _final_body(p0_ref, p1_ref, din_ref, b_ref, wm_ref, wa_ref, o_ref):
    ndst = lax.rsqrt(jnp.maximum(din_ref[:, 0:1], 1.0))
    t = jnp.maximum((p0_ref[...] + p1_ref[...]) * ndst + b_ref[...], 0.0)
    mx = jnp.max(t, axis=0, keepdims=True)
    sm = jnp.sum(t, axis=0, keepdims=True)
    o_ref[...] = wm_ref[...] * mx + (wa_ref[...] / N) * sm


def _tc_final(p0, p1, deg_in, b, wm, wa):
    return pl.pallas_call(
        _final_body,
        out_shape=jax.ShapeDtypeStruct((1, D), jnp.float32),
    )(p0, p1, deg_in, b, wm, wa)


def kernel(features, edge_index, W0, b0, W1, b1, W2, b2, pool_weight):
    src = edge_index[0]
    dst = edge_index[1]
    npad = E_PAD - E
    pad_sink = jnp.full((npad,), N, dtype=jnp.int32)
    pad_zero = jnp.zeros((npad,), dtype=jnp.int32)
    src_r = jnp.concatenate([src, pad_zero]).reshape(NC * NS, NCH, CH)
    dst_r = jnp.concatenate([dst, pad_sink]).reshape(NC * NS, NCH, CH)
    idx2 = jnp.concatenate(
        [src, pad_sink, dst, pad_sink]).reshape(2 * NS, DCH, CHD)
    ones128 = jnp.ones((CHD, D), jnp.float32)
    zeros128 = jnp.zeros((RPT, D), jnp.float32)

    deg = _sc_degree(idx2, zeros128, ones128)
    deg_out = deg[:N]
    deg_in = deg[N_PAD:N_PAD + N]

    hn = _tc_first(features, W0, deg_out)
    part = _sc_scatter(hn, src_r, dst_r, zeros128)
    p0, p1 = part[:N], part[N_PAD:N_PAD + N]
    hn = _tc_mid(p0, p1, deg_in, deg_out, b0.reshape(1, D), W1)
    part = _sc_scatter(hn, src_r, dst_r, zeros128)
    p0, p1 = part[:N], part[N_PAD:N_PAD + N]
    hn = _tc_mid(p0, p1, deg_in, deg_out, b1.reshape(1, D), W2)
    part = _sc_scatter(hn, src_r, dst_r, zeros128)
    p0, p1 = part[:N], part[N_PAD:N_PAD + N]

    w = jax.nn.softmax(pool_weight, axis=0)
    return _tc_final(
        p0, p1, deg_in, b2.reshape(1, D),
        w[0].reshape(1, 1), w[1].reshape(1, 1),
    )


# speedup vs baseline: 1.2640x; 1.2636x over previous
import functools

import jax
import jax.numpy as jnp
from jax import lax
from jax.experimental import pallas as pl
from jax.experimental.pallas import tpu as pltpu
from jax.experimental.pallas import tpu_sc as plsc

N = 10000
D = 128
E = 320000
NC = 2
NS = 16
CH = 128
TCH = 160
SPL = 120
DMAX = 120
E_PAD = NS * TCH * CH
N_PAD = 10112
RPT = N_PAD // NS
DCH = 160
CHD = 128


def _sc_mesh():
    return plsc.VectorSubcoreMesh(core_axis_name="c", subcore_axis_name="s")


def _sc_degree(idx2, zeros128, ones128):
    @functools.partial(
        pl.kernel,
        mesh=_sc_mesh(),
        out_type=jax.ShapeDtypeStruct((2 * N_PAD, D), jnp.float32),
        scratch_types=[
            pltpu.VMEM((DCH, CHD), jnp.int32),
            pltpu.VMEM((CHD, D), jnp.float32),
            pltpu.VMEM_SHARED((N_PAD, D), jnp.float32),
        ],
    )
    def k(idx_hbm, z_hbm, o_hbm, deg_hbm, idx_all, ones_v, acc):
        c = lax.axis_index("c")
        s = lax.axis_index("s")
        pltpu.sync_copy(z_hbm, acc.at[pl.ds(s * RPT, RPT)])
        pltpu.sync_copy(o_hbm, ones_v)
        pltpu.sync_copy(idx_hbm.at[c * NS + s], idx_all)
        plsc.subcore_barrier()

        def body(g, carry):
            pltpu.sync_copy(ones_v, acc.at[idx_all.at[g]], add=True)
            return carry

        lax.fori_loop(0, DCH, body, 0)
        plsc.subcore_barrier()
        pltpu.sync_copy(
            acc.at[pl.ds(s * RPT, RPT)],
            deg_hbm.at[pl.ds(c * N_PAD + s * RPT, RPT)],
        )

    return k(idx2, zeros128, ones128)


def _sc_scatter(hn, src_r, dst_r, zeros128):
    @functools.partial(
        pl.kernel,
        mesh=_sc_mesh(),
        out_type=jax.ShapeDtypeStruct((2 * N_PAD, D), jnp.float32),
        scratch_types=[
            pltpu.VMEM((DMAX, CH), jnp.int32),
            pltpu.VMEM((CH,), jnp.int32),
            pltpu.VMEM((CH,), jnp.int32),
            pltpu.VMEM((CH, D), jnp.float32),
            pltpu.VMEM((CH, D), jnp.float32),
            pltpu.SemaphoreType.DMA,
            pltpu.SemaphoreType.DMA,
            pltpu.SemaphoreType.DMA,
            pltpu.SemaphoreType.DMA,
            pltpu.VMEM_SHARED((N_PAD, D), jnp.float32),
        ],
    )
    def k(hn_hbm, src_hbm, dst_hbm, z_hbm, out_hbm,
          didx, ibuf0, ibuf1, rbuf0, rbuf1,
          isem0, isem1, rsem0, rsem1, acc):
        c = lax.axis_index("c")
        s = lax.axis_index("s")
        n = jnp.where(c == 0, SPL, TCH - SPL)
        cbase = c * SPL
        lbase = c * (SPL - (TCH - DMAX))
        pltpu.sync_copy(z_hbm, acc.at[pl.ds(s * RPT, RPT)])
        pltpu.sync_copy(dst_hbm.at[s, pl.ds(c * (TCH - DMAX), DMAX)], didx)
        plsc.subcore_barrier()

        def idx_start(g, ibuf, isem):
            pltpu.async_copy(src_hbm.at[s, cbase + g], ibuf, isem)

        def idx_wait(ibuf, isem):
            pltpu.make_async_copy(src_hbm.at[s, 0], ibuf, isem).wait()

        def gather_start(ibuf, rbuf, rsem):
            pltpu.async_copy(hn_hbm.at[ibuf], rbuf, rsem)

        def gather_wait(ibuf, rbuf, rsem):
            pltpu.make_async_copy(hn_hbm.at[ibuf], rbuf, rsem).wait()

        def scat(g, rbuf):
            pltpu.sync_copy(rbuf, acc.at[didx.at[lbase + g]], add=True)

        idx_start(0, ibuf0, isem0)
        idx_start(1, ibuf1, isem1)
        idx_wait(ibuf0, isem0)
        gather_start(ibuf0, rbuf0, rsem0)

        def body(i, carry):
            g = 2 * i
            idx_wait(ibuf1, isem1)
            gather_wait(ibuf0, rbuf0, rsem0)
            gather_start(ibuf1, rbuf1, rsem1)
            idx_start(g + 2, ibuf0, isem0)
            scat(g, rbuf0)
            idx_wait(ibuf0, isem0)
            gather_wait(ibuf1, rbuf1, rsem1)
            gather_start(ibuf0, rbuf0, rsem0)
            idx_start(g + 3, ibuf1, isem1)
            scat(g + 1, rbuf1)
            return carry

        lax.fori_loop(0, n // 2 - 1, body, 0)
        gl = n - 2
        idx_wait(ibuf1, isem1)
        gather_wait(ibuf0, rbuf0, rsem0)
        gather_start(ibuf1, rbuf1, rsem1)
        scat(gl, rbuf0)
        gather_wait(ibuf1, rbuf1, rsem1)
        scat(gl + 1, rbuf1)

        plsc.subcore_barrier()
        pltpu.sync_copy(
            acc.at[pl.ds(s * RPT, RPT)],
            out_hbm.at[pl.ds(c * N_PAD + s * RPT, RPT)],
        )

    return k(hn, src_r, dst_r, zeros128)


def _first_body(x_ref, w_ref, deg_ref, o_ref):
    nsrc = lax.rsqrt(jnp.maximum(deg_ref[:, 0:1], 1.0))
    o_ref[...] = (
        jnp.dot(x_ref[...], w_ref[...], preferred_element_type=jnp.float32)
        * nsrc
    )


def _tc_first(x, w, deg_out):
    return pl.pallas_call(
        _first_body,
        out_shape=jax.ShapeDtypeStruct((N, D), jnp.float32),
    )(x, w, deg_out)


def _mid_body(p0_ref, p1_ref, din_ref, dout_ref, b_ref, w_ref, o_ref):
    ndst = lax.rsqrt(jnp.maximum(din_ref[:, 0:1], 1.0))
    t = jnp.maximum((p0_ref[...] + p1_ref[...]) * ndst + b_ref[...], 0.0)
    nsrc = lax.rsqrt(jnp.maximum(dout_ref[:, 0:1], 1.0))
    o_ref[...] = (
        jnp.dot(t, w_ref[...], preferred_element_type=jnp.float32) * nsrc
    )


def _tc_mid(p0, p1, deg_in, deg_out, b, w):
    return pl.pallas_call(
        _mid_body,
        out_shape=jax.ShapeDtypeStruct((N, D), jnp.float32),
    )(p0, p1, deg_in, deg_out, b, w)


def _final_body(p0_ref, p1_ref, din_ref, b_ref, wm_ref, wa_ref, o_ref):
    ndst = lax.rsqrt(jnp.maximum(din_ref[:, 0:1], 1.0))
    t = jnp.maximum((p0_ref[...] + p1_ref[...]) * ndst + b_ref[...], 0.0)
    mx = jnp.max(t, axis=0, keepdims=True)
    sm = jnp.sum(t, axis=0, keepdims=True)
    o_ref[...] = wm_ref[...] * mx + (wa_ref[...] / N) * sm


def _tc_final(p0, p1, deg_in, b, wm, wa):
    return pl.pallas_call(
        _final_body,
        out_shape=jax.ShapeDtypeStruct((1, D), jnp.float32),
    )(p0, p1, deg_in, b, wm, wa)


def kernel(features, edge_index, W0, b0, W1, b1, W2, b2, pool_weight):
    src = edge_index[0]
    dst = edge_index[1]
    npad = E_PAD - E
    pad_sink = jnp.full((npad,), N, dtype=jnp.int32)
    pad_zero = jnp.zeros((npad,), dtype=jnp.int32)
    src_r = jnp.concatenate([src, pad_zero]).reshape(NS, TCH, CH)
    dst_r = jnp.concatenate([dst, pad_sink]).reshape(NS, TCH, CH)
    idx2 = jnp.concatenate(
        [src, pad_sink, dst, pad_sink]).reshape(2 * NS, DCH, CHD)
    ones128 = jnp.ones((CHD, D), jnp.float32)
    zeros128 = jnp.zeros((RPT, D), jnp.float32)

    deg = _sc_degree(idx2, zeros128, ones128)
    deg_out = deg[:N]
    deg_in = deg[N_PAD:N_PAD + N]

    hn = _tc_first(features, W0, deg_out)
    part = _sc_scatter(hn, src_r, dst_r, zeros128)
    p0, p1 = part[:N], part[N_PAD:N_PAD + N]
    hn = _tc_mid(p0, p1, deg_in, deg_out, b0.reshape(1, D), W1)
    part = _sc_scatter(hn, src_r, dst_r, zeros128)
    p0, p1 = part[:N], part[N_PAD:N_PAD + N]
    hn = _tc_mid(p0, p1, deg_in, deg_out, b1.reshape(1, D), W2)
    part = _sc_scatter(hn, src_r, dst_r, zeros128)
    p0, p1 = part[:N], part[N_PAD:N_PAD + N]

    w = jax.nn.softmax(pool_weight, axis=0)
    return _tc_final(
        p0, p1, deg_in, b2.reshape(1, D),
        w[0].reshape(1, 1), w[1].reshape(1, 1),
    )
